# baseline (speedup 1.0000x reference)
import functools

import jax
import jax.numpy as jnp
from jax import lax
from jax.experimental import pallas as pl
from jax.experimental.pallas import tpu as pltpu
from jax.experimental.pallas import tpu_sc as plsc

N = 100000
D = 128
DD = 16
NW = 32
NPAD = 100352
CPW = NPAD // NW
SUB = 112
NSUB = CPW // SUB
NROW = NPAD // 8
NCH = NPAD // 8
BLKR = 448
NBR = NROW // BLKR
BLKF = 1792
NBF = NCH // BLKF


_SC_PARAMS = pltpu.CompilerParams(use_tc_tiling_on_sc=False)


def _proj_body(z0, z1, z2, z3, z4, z5, z6, z7, w_ref, b_ref, p0, p1, p2, p3):
    w = w_ref[...]
    accs = [
        jnp.dot(zg[...], w, preferred_element_type=jnp.float32)
        for zg in (z0, z1, z2, z3, z4, z5, z6, z7)
    ]
    rows = jax.lax.broadcasted_iota(jnp.int32, (BLKR, 1), 0) \
        + pl.program_id(0) * BLKR
    accs[7] = jnp.where(rows < N - 7 * NCH, accs[7], 0.0)
    p0[...] = jnp.concatenate([a[:, 0:16] for a in accs], axis=1) + b_ref[...]
    p1[...] = jnp.concatenate([a[:, 16:32] for a in accs], axis=1)
    p2[...] = jnp.concatenate([a[:, 32:48] for a in accs], axis=1)
    p3[...] = jnp.concatenate([a[:, 48:64] for a in accs], axis=1)


def _project(z, wstack, b128):
    out = jax.ShapeDtypeStruct((NROW, D), jnp.float32)
    zspecs = [
        pl.BlockSpec((BLKR, D), lambda r, g=g: (g * NBR + r, 0))
        for g in range(8)
    ]
    return pl.pallas_call(
        _proj_body,
        grid=(NBR,),
        in_specs=zspecs + [
            pl.BlockSpec((D, 4 * DD), lambda r: (0, 0)),
            pl.BlockSpec((1, D), lambda r: (0, 0)),
        ],
        out_specs=[pl.BlockSpec((BLKR, D), lambda r: (r, 0))] * 4,
        out_shape=[out] * 4,
    )(*([z] * 8), wstack, b128)


def _final_body(z_ref, f1_ref, h1_ref, h2_ref, h3_ref, k0_ref, k1_ref,
                k2_ref, k3_ref, out_ref, s_ref):
    g = pl.program_id(1)

    @pl.when(g == 0)
    def _():
        f1 = f1_ref[...]
        s_ref[...] = (
            2.0 * f1
            + jnp.dot(f1, k0_ref[...], preferred_element_type=jnp.float32)
            + jnp.dot(h1_ref[...], k1_ref[...],
                      preferred_element_type=jnp.float32)
            + jnp.dot(h2_ref[...], k2_ref[...],
                      preferred_element_type=jnp.float32)
            + jnp.dot(h3_ref[...], k3_ref[...],
                      preferred_element_type=jnp.float32))

    s16 = lax.switch(
        g, [lambda gg=gg: s_ref[:, gg * DD:(gg + 1) * DD] for gg in range(8)])
    zz = z_ref[...]
    out_ref[...] = jnp.concatenate([zz[:, :DD] + s16, zz[:, DD:]], axis=1)


def _final(z, f1, h1, h2, h3, kmats):
    fb = pl.BlockSpec((BLKF, D), lambda r, g: (r, 0))
    zb = pl.BlockSpec((BLKF, D), lambda r, g: (g * NBF + r, 0))
    kb = pl.BlockSpec((D, D), lambda r, g: (0, 0))
    return pl.pallas_call(
        _final_body,
        grid=(NBF, 8),
        in_specs=[zb, fb, fb, fb, fb, kb, kb, kb, kb],
        out_specs=zb,
        out_shape=jax.ShapeDtypeStruct((N, D), jnp.float32),
        scratch_shapes=[pltpu.VMEM((BLKF, D), jnp.float32)],
    )(z, f1, h1, h2, h3, *kmats)


def _gather_sum(p0, p1, p2, p3, nl3):
    mesh = plsc.VectorSubcoreMesh(core_axis_name="c", subcore_axis_name="s")

    @functools.partial(
        pl.kernel, mesh=mesh, compiler_params=_SC_PARAMS,
        out_type=jax.ShapeDtypeStruct((NPAD, DD), jnp.float32),
        scratch_types=[
            pltpu.VMEM((3, NSUB, SUB), jnp.int32),
            pltpu.VMEM((CPW, DD), jnp.float32),
            pltpu.VMEM((SUB, DD), jnp.float32),
            pltpu.VMEM((SUB, DD), jnp.float32),
            pltpu.VMEM((SUB, DD), jnp.float32),
            pltpu.VMEM((SUB, DD), jnp.float32),
            pltpu.SemaphoreType.DMA,
        ],
    )
    def body(p0_h, p1_h, p2_h, p3_h, nl_h, f_h, idx_v, p0_v, r1, r2, r3, fb,
             sem):
        wid = lax.axis_index("s") * 2 + lax.axis_index("c")
        base_w = pl.multiple_of(wid * CPW, CPW)
        pltpu.sync_copy(nl_h.at[wid], idx_v)
        pltpu.sync_copy(p0_h.at[pl.ds(base_w, CPW)], p0_v)

        def sub(s, carry):
            g1 = pltpu.async_copy(p1_h.at[idx_v.at[0, s]], r1, sem)
            g2 = pltpu.async_copy(p2_h.at[idx_v.at[1, s]], r2, sem)
            g3 = pltpu.async_copy(p3_h.at[idx_v.at[2, s]], r3, sem)
            g1.wait()
            g2.wait()
            g3.wait()
            rowbase = s * SUB

            def row(i, c):
                fb[i] = (p0_v[rowbase + i] + r1[i]) + (r2[i] + r3[i])
                return c

            lax.fori_loop(0, SUB, row, 0)
            off = pl.multiple_of(base_w + rowbase, SUB)
            pltpu.sync_copy(fb, f_h.at[pl.ds(off, SUB)])
            return carry

        lax.fori_loop(0, NSUB, sub, 0)

    return body(p0, p1, p2, p3, nl3)


def _gather3(f1, nl3):
    mesh = plsc.VectorSubcoreMesh(core_axis_name="c", subcore_axis_name="s")
    ht = jax.ShapeDtypeStruct((NPAD, DD), jnp.float32)

    @functools.partial(
        pl.kernel, mesh=mesh, compiler_params=_SC_PARAMS,
        out_type=[ht, ht, ht],
        scratch_types=[
            pltpu.VMEM((3, NSUB, SUB), jnp.int32),
            pltpu.VMEM((SUB, DD), jnp.float32),
            pltpu.VMEM((SUB, DD), jnp.float32),
            pltpu.VMEM((SUB, DD), jnp.float32),
            pltpu.SemaphoreType.DMA,
        ],
    )
    def body(f1_h, nl_h, h1_h, h2_h, h3_h, idx_v, r1, r2, r3, sem):
        wid = lax.axis_index("s") * 2 + lax.axis_index("c")
        base_w = pl.multiple_of(wid * CPW, CPW)
        pltpu.sync_copy(nl_h.at[wid], idx_v)

        def sub(s, carry):
            g1 = pltpu.async_copy(f1_h.at[idx_v.at[0, s]], r1, sem)
            g2 = pltpu.async_copy(f1_h.at[idx_v.at[1, s]], r2, sem)
            g3 = pltpu.async_copy(f1_h.at[idx_v.at[2, s]], r3, sem)
            off = pl.multiple_of(base_w + s * SUB, SUB)
            g1.wait()
            pltpu.sync_copy(r1, h1_h.at[pl.ds(off, SUB)])
            g2.wait()
            pltpu.sync_copy(r2, h2_h.at[pl.ds(off, SUB)])
            g3.wait()
            pltpu.sync_copy(r3, h3_h.at[pl.ds(off, SUB)])
            return carry

        lax.fori_loop(0, NSUB, sub, 0)

    return body(f1, nl3)


def kernel(z_old, W, b, neighbour_list):
    w0, w1, w2, w3 = W[0::4], W[1::4], W[2::4], W[3::4]
    wstack = jnp.concatenate([w0, w1, w2, w3], axis=1)
    b128 = jnp.tile(b, 8).reshape(1, D)
    eye8 = jnp.eye(8, dtype=jnp.float32)
    kmats = [jnp.kron(eye8, wk[:DD]) for wk in (w0, w1, w2, w3)]
    tt = jnp.arange(NPAD, dtype=jnp.int32)
    p_of_t = tt // 8 + (tt % 8) * NCH
    p_safe = jnp.where(p_of_t < N, p_of_t, 0)
    nl_t = neighbour_list[p_safe]
    nl_t = (nl_t % NCH) * 8 + nl_t // NCH
    nl3 = jnp.transpose(nl_t.T.reshape(3, NW, NSUB, SUB), (1, 0, 2, 3))

    flat = lambda a: jnp.reshape(a, (NPAD, DD))
    wide = lambda a: jnp.reshape(a, (NROW, D))
    p0, p1, p2, p3 = _project(z_old, wstack, b128)
    f1 = _gather_sum(flat(p0), flat(p1), flat(p2), flat(p3), nl3)
    h1, h2, h3 = _gather3(f1, nl3)
    return _final(z_old, wide(f1), wide(h1), wide(h2), wide(h3), kmats)


# speedup vs baseline: 1.5953x; 1.5953x over previous
import functools

import jax
import jax.numpy as jnp
from jax import lax
from jax.experimental import pallas as pl
from jax.experimental.pallas import tpu as pltpu
from jax.experimental.pallas import tpu_sc as plsc

N = 100000
D = 128
DD = 16
NW = 32
NPAD = 100352
CPW = NPAD // NW
SUB = 112
NSUB = CPW // SUB
NROW = NPAD // 8
BLKR = 448
NBR = NROW // BLKR
PBLK = 8 * BLKR
FBW = 2
BLKF = FBW * BLKR
NBF = NROW // BLKF


_SC_PARAMS = pltpu.CompilerParams(use_tc_tiling_on_sc=False)


def _proj_body(z_ref, w_ref, b_ref, p0, p1, p2, p3):
    w = w_ref[...]
    zz = z_ref[...]
    accs = [
        jnp.dot(zz[g * BLKR:(g + 1) * BLKR, :], w,
                preferred_element_type=jnp.float32)
        for g in range(8)
    ]
    rows = jax.lax.broadcasted_iota(jnp.int32, (BLKR, 1), 0)
    limit = N - PBLK * pl.program_id(0) - 7 * BLKR
    accs[7] = jnp.where(rows < limit, accs[7], 0.0)
    p0[...] = jnp.concatenate([a[:, 0:16] for a in accs], axis=1) + b_ref[...]
    p1[...] = jnp.concatenate([a[:, 16:32] for a in accs], axis=1)
    p2[...] = jnp.concatenate([a[:, 32:48] for a in accs], axis=1)
    p3[...] = jnp.concatenate([a[:, 48:64] for a in accs], axis=1)


def _project(z, wstack, b128):
    out = jax.ShapeDtypeStruct((NROW, D), jnp.float32)
    return pl.pallas_call(
        _proj_body,
        grid=(NBR,),
        in_specs=[
            pl.BlockSpec((PBLK, D), lambda r: (r, 0)),
            pl.BlockSpec((D, 4 * DD), lambda r: (0, 0)),
            pl.BlockSpec((1, D), lambda r: (0, 0)),
        ],
        out_specs=[pl.BlockSpec((BLKR, D), lambda r: (r, 0))] * 4,
        out_shape=[out] * 4,
    )(z, wstack, b128)


def _final_body(z_ref, f1_ref, h1_ref, h2_ref, h3_ref, k0_ref, k1_ref,
                k2_ref, k3_ref, out_ref):
    f1 = f1_ref[...]
    s = (2.0 * f1
         + jnp.dot(f1, k0_ref[...], preferred_element_type=jnp.float32)
         + jnp.dot(h1_ref[...], k1_ref[...], preferred_element_type=jnp.float32)
         + jnp.dot(h2_ref[...], k2_ref[...], preferred_element_type=jnp.float32)
         + jnp.dot(h3_ref[...], k3_ref[...], preferred_element_type=jnp.float32))
    zz = z_ref[...]
    pieces = []
    for blk in range(FBW):
        for g in range(8):
            zrows = zz[blk * PBLK + g * BLKR:blk * PBLK + (g + 1) * BLKR, :]
            s16 = s[blk * BLKR:(blk + 1) * BLKR, g * DD:(g + 1) * DD]
            pieces.append(
                jnp.concatenate([zrows[:, :DD] + s16, zrows[:, DD:]], axis=1))
    out_ref[...] = jnp.concatenate(pieces, axis=0)


def _final(z, f1, h1, h2, h3, kmats):
    fb = pl.BlockSpec((BLKF, D), lambda r: (r, 0))
    zb = pl.BlockSpec((FBW * PBLK, D), lambda r: (r, 0))
    kb = pl.BlockSpec((D, D), lambda r: (0, 0))
    return pl.pallas_call(
        _final_body,
        grid=(NBF,),
        in_specs=[zb, fb, fb, fb, fb, kb, kb, kb, kb],
        out_specs=zb,
        out_shape=jax.ShapeDtypeStruct((N, D), jnp.float32),
    )(z, f1, h1, h2, h3, *kmats)


def _gather_sum(p0, p1, p2, p3, nl3):
    mesh = plsc.VectorSubcoreMesh(core_axis_name="c", subcore_axis_name="s")
    rbuf = pltpu.VMEM((SUB, DD), jnp.float32)

    @functools.partial(
        pl.kernel, mesh=mesh, compiler_params=_SC_PARAMS,
        out_type=jax.ShapeDtypeStruct((NPAD, DD), jnp.float32),
        scratch_types=[
            pltpu.VMEM((3, NSUB, SUB), jnp.int32),
            pltpu.VMEM((CPW, DD), jnp.float32),
            rbuf, rbuf, rbuf, rbuf, rbuf, rbuf, rbuf, rbuf,
            pltpu.SemaphoreType.DMA,
            pltpu.SemaphoreType.DMA,
        ],
    )
    def body(p0_h, p1_h, p2_h, p3_h, nl_h, f_h, idx_v, p0_v,
             ra1, ra2, ra3, rb1, rb2, rb3, fba, fbb, sema, semb):
        wid = lax.axis_index("s") * 2 + lax.axis_index("c")
        base_w = pl.multiple_of(wid * CPW, CPW)
        pltpu.sync_copy(nl_h.at[wid], idx_v)
        pltpu.sync_copy(p0_h.at[pl.ds(base_w, CPW)], p0_v)
        bufs_a, bufs_b = (ra1, ra2, ra3), (rb1, rb2, rb3)
        tabs = (p1_h, p2_h, p3_h)

        def issue(s, bufs, sem):
            for k in range(3):
                pltpu.async_copy(tabs[k].at[idx_v.at[k, s]], bufs[k], sem)

        def waitall(s, bufs, sem):
            for k in range(3):
                pltpu.make_async_copy(
                    tabs[k].at[idx_v.at[k, s]], bufs[k], sem).wait()

        def compute(s, bufs, fb):
            rowbase = s * SUB
            r1, r2, r3 = bufs

            def row4(i4, c):
                i = i4 * 4
                for u in range(4):
                    fb[i + u] = ((p0_v[rowbase + i + u] + r1[i + u])
                                 + (r2[i + u] + r3[i + u]))
                return c

            lax.fori_loop(0, SUB // 4, row4, 0)
            off = pl.multiple_of(base_w + rowbase, SUB)
            pltpu.sync_copy(fb, f_h.at[pl.ds(off, SUB)])

        issue(0, bufs_a, sema)

        def step(k, c):
            s = 2 * k
            issue(s + 1, bufs_b, semb)
            waitall(s, bufs_a, sema)
            compute(s, bufs_a, fba)

            @pl.when(s + 2 < NSUB)
            def _():
                issue(s + 2, bufs_a, sema)

            waitall(s + 1, bufs_b, semb)
            compute(s + 1, bufs_b, fbb)
            return c

        lax.fori_loop(0, NSUB // 2, step, 0)

    return body(p0, p1, p2, p3, nl3)


def _gather3(f1, nl3):
    mesh = plsc.VectorSubcoreMesh(core_axis_name="c", subcore_axis_name="s")
    ht = jax.ShapeDtypeStruct((NPAD, DD), jnp.float32)
    rbuf = pltpu.VMEM((SUB, DD), jnp.float32)

    @functools.partial(
        pl.kernel, mesh=mesh, compiler_params=_SC_PARAMS,
        out_type=[ht, ht, ht],
        scratch_types=[
            pltpu.VMEM((3, NSUB, SUB), jnp.int32),
            rbuf, rbuf, rbuf, rbuf, rbuf, rbuf,
            pltpu.SemaphoreType.DMA,
            pltpu.SemaphoreType.DMA,
        ],
    )
    def body(f1_h, nl_h, h1_h, h2_h, h3_h, idx_v,
             ra1, ra2, ra3, rb1, rb2, rb3, sema, semb):
        wid = lax.axis_index("s") * 2 + lax.axis_index("c")
        base_w = pl.multiple_of(wid * CPW, CPW)
        pltpu.sync_copy(nl_h.at[wid], idx_v)
        bufs_a, bufs_b = (ra1, ra2, ra3), (rb1, rb2, rb3)
        outs = (h1_h, h2_h, h3_h)

        def issue(s, bufs, sem):
            for k in range(3):
                pltpu.async_copy(f1_h.at[idx_v.at[k, s]], bufs[k], sem)

        def drain(s, bufs, sem):
            off = pl.multiple_of(base_w + s * SUB, SUB)
            for k in range(3):
                pltpu.make_async_copy(
                    f1_h.at[idx_v.at[k, s]], bufs[k], sem).wait()
                pltpu.sync_copy(bufs[k], outs[k].at[pl.ds(off, SUB)])

        issue(0, bufs_a, sema)

        def step(k, c):
            s = 2 * k
            issue(s + 1, bufs_b, semb)
            drain(s, bufs_a, sema)

            @pl.when(s + 2 < NSUB)
            def _():
                issue(s + 2, bufs_a, sema)

            drain(s + 1, bufs_b, semb)
            return c

        lax.fori_loop(0, NSUB // 2, step, 0)

    return body(f1, nl3)


def kernel(z_old, W, b, neighbour_list):
    w0, w1, w2, w3 = W[0::4], W[1::4], W[2::4], W[3::4]
    wstack = jnp.concatenate([w0, w1, w2, w3], axis=1)
    b128 = jnp.tile(b, 8).reshape(1, D)
    eye8 = jnp.eye(8, dtype=jnp.float32)
    kmats = [jnp.kron(eye8, wk[:DD]) for wk in (w0, w1, w2, w3)]
    q = neighbour_list % PBLK
    nlv = 8 * (BLKR * (neighbour_list // PBLK) + q % BLKR) + q // BLKR
    nlv = jnp.zeros((NPAD, 3), jnp.int32).at[:N].set(nlv)
    nl_t = jnp.transpose(nlv.reshape(NBR, 8, BLKR, 3), (0, 2, 1, 3))
    nl_t = nl_t.reshape(NPAD, 3)
    nl3 = jnp.transpose(nl_t.T.reshape(3, NW, NSUB, SUB), (1, 0, 2, 3))

    flat = lambda a: jnp.reshape(a, (NPAD, DD))
    wide = lambda a: jnp.reshape(a, (NROW, D))
    p0, p1, p2, p3 = _project(z_old, wstack, b128)
    f1 = _gather_sum(flat(p0), flat(p1), flat(p2), flat(p3), nl3)
    h1, h2, h3 = _gather3(f1, nl3)
    return _final(z_old, wide(f1), wide(h1), wide(h2), wide(h3), kmats)


# speedup vs baseline: 1.7446x; 1.0936x over previous
import functools

import jax
import jax.numpy as jnp
from jax import lax
from jax.experimental import pallas as pl
from jax.experimental.pallas import tpu as pltpu
from jax.experimental.pallas import tpu_sc as plsc

N = 100000
D = 128
DD = 16
NW = 32
NPAD = 100352
CPW = NPAD // NW
SUB = 112
NSUB = CPW // SUB
NROW = NPAD // 8
BLKR = 392
NBR = NROW // BLKR
PBLK = 8 * BLKR
FBW = 2
BLKF = FBW * BLKR
NBF = NROW // BLKF


_SC_PARAMS = pltpu.CompilerParams(use_tc_tiling_on_sc=False)


def _proj_body(z_ref, w_ref, b_ref, p0, p1, p2, p3):
    w2 = w_ref[...]
    zz = z_ref[...]
    rows = jax.lax.broadcasted_iota(jnp.int32, (PBLK, 1), 0)
    zz = jnp.where(rows < N - PBLK * pl.program_id(0), zz, 0.0)
    accs = []
    for j in range(4):
        z2 = jnp.concatenate(
            [zz[2 * j * BLKR:(2 * j + 1) * BLKR, :],
             zz[(2 * j + 1) * BLKR:(2 * j + 2) * BLKR, :]], axis=1)
        acc2 = jnp.dot(z2, w2, preferred_element_type=jnp.float32)
        accs.append(acc2[:, 0:64])
        accs.append(acc2[:, 64:128])
    p0[...] = jnp.concatenate([a[:, 0:16] for a in accs], axis=1) + b_ref[...]
    p1[...] = jnp.concatenate([a[:, 16:32] for a in accs], axis=1)
    p2[...] = jnp.concatenate([a[:, 32:48] for a in accs], axis=1)
    p3[...] = jnp.concatenate([a[:, 48:64] for a in accs], axis=1)


def _project(z, wstack, b128):
    out = jax.ShapeDtypeStruct((NROW, D), jnp.float32)
    return pl.pallas_call(
        _proj_body,
        grid=(NBR,),
        in_specs=[
            pl.BlockSpec((PBLK, D), lambda r: (r, 0)),
            pl.BlockSpec((2 * D, D), lambda r: (0, 0)),
            pl.BlockSpec((1, D), lambda r: (0, 0)),
        ],
        out_specs=[pl.BlockSpec((BLKR, D), lambda r: (r, 0))] * 4,
        out_shape=[out] * 4,
    )(z, wstack, b128)


def _final_body(z_ref, f1_ref, h1_ref, h2_ref, h3_ref, k0_ref, k1_ref,
                k2_ref, k3_ref, out_ref):
    f1 = f1_ref[...]
    s = (2.0 * f1
         + jnp.dot(f1, k0_ref[...], preferred_element_type=jnp.float32)
         + jnp.dot(h1_ref[...], k1_ref[...], preferred_element_type=jnp.float32)
         + jnp.dot(h2_ref[...], k2_ref[...], preferred_element_type=jnp.float32)
         + jnp.dot(h3_ref[...], k3_ref[...], preferred_element_type=jnp.float32))
    zz = z_ref[...]
    pieces = []
    for blk in range(FBW):
        for g in range(8):
            zrows = zz[blk * PBLK + g * BLKR:blk * PBLK + (g + 1) * BLKR, :]
            s16 = s[blk * BLKR:(blk + 1) * BLKR, g * DD:(g + 1) * DD]
            pieces.append(
                jnp.concatenate([zrows[:, :DD] + s16, zrows[:, DD:]], axis=1))
    out_ref[...] = jnp.concatenate(pieces, axis=0)


def _final(z, f1, h1, h2, h3, kmats):
    fb = pl.BlockSpec((BLKF, D), lambda r: (r, 0))
    zb = pl.BlockSpec((FBW * PBLK, D), lambda r: (r, 0))
    kb = pl.BlockSpec((D, D), lambda r: (0, 0))
    return pl.pallas_call(
        _final_body,
        grid=(NBF,),
        in_specs=[zb, fb, fb, fb, fb, kb, kb, kb, kb],
        out_specs=zb,
        out_shape=jax.ShapeDtypeStruct((N, D), jnp.float32),
    )(z, f1, h1, h2, h3, *kmats)


def _gather_sum(p0, p1, p2, p3, nl3):
    mesh = plsc.VectorSubcoreMesh(core_axis_name="c", subcore_axis_name="s")
    rbuf = pltpu.VMEM((SUB, DD), jnp.float32)

    @functools.partial(
        pl.kernel, mesh=mesh, compiler_params=_SC_PARAMS,
        out_type=jax.ShapeDtypeStruct((NPAD, DD), jnp.float32),
        scratch_types=[
            pltpu.VMEM((3, NSUB, SUB), jnp.int32),
            pltpu.VMEM((CPW, DD), jnp.float32),
            rbuf, rbuf, rbuf, rbuf, rbuf, rbuf, rbuf, rbuf,
            pltpu.SemaphoreType.DMA,
            pltpu.SemaphoreType.DMA,
        ],
    )
    def body(p0_h, p1_h, p2_h, p3_h, nl_h, f_h, idx_v, p0_v,
             ra1, ra2, ra3, rb1, rb2, rb3, fba, fbb, sema, semb):
        wid = lax.axis_index("s") * 2 + lax.axis_index("c")
        base_w = pl.multiple_of(wid * CPW, CPW)
        pltpu.sync_copy(nl_h.at[wid], idx_v)
        pltpu.sync_copy(p0_h.at[pl.ds(base_w, CPW)], p0_v)
        bufs_a, bufs_b = (ra1, ra2, ra3), (rb1, rb2, rb3)
        tabs = (p1_h, p2_h, p3_h)

        def issue(s, bufs, sem):
            for k in range(3):
                pltpu.async_copy(tabs[k].at[idx_v.at[k, s]], bufs[k], sem)

        def waitall(s, bufs, sem):
            for k in range(3):
                pltpu.make_async_copy(
                    tabs[k].at[idx_v.at[k, s]], bufs[k], sem).wait()

        def compute(s, bufs, fb):
            rowbase = s * SUB
            r1, r2, r3 = bufs

            def row4(i4, c):
                i = i4 * 4
                for u in range(4):
                    fb[i + u] = ((p0_v[rowbase + i + u] + r1[i + u])
                                 + (r2[i + u] + r3[i + u]))
                return c

            lax.fori_loop(0, SUB // 4, row4, 0)
            off = pl.multiple_of(base_w + rowbase, SUB)
            pltpu.sync_copy(fb, f_h.at[pl.ds(off, SUB)])

        issue(0, bufs_a, sema)

        def step(k, c):
            s = 2 * k
            issue(s + 1, bufs_b, semb)
            waitall(s, bufs_a, sema)
            compute(s, bufs_a, fba)

            @pl.when(s + 2 < NSUB)
            def _():
                issue(s + 2, bufs_a, sema)

            waitall(s + 1, bufs_b, semb)
            compute(s + 1, bufs_b, fbb)
            return c

        lax.fori_loop(0, NSUB // 2, step, 0)

    return body(p0, p1, p2, p3, nl3)


def _gather3(f1, nl3):
    mesh = plsc.VectorSubcoreMesh(core_axis_name="c", subcore_axis_name="s")
    ht = jax.ShapeDtypeStruct((NPAD, DD), jnp.float32)
    rbuf = pltpu.VMEM((SUB, DD), jnp.float32)

    @functools.partial(
        pl.kernel, mesh=mesh, compiler_params=_SC_PARAMS,
        out_type=[ht, ht, ht],
        scratch_types=[
            pltpu.VMEM((3, NSUB, SUB), jnp.int32),
            rbuf, rbuf, rbuf, rbuf, rbuf, rbuf,
            pltpu.SemaphoreType.DMA,
            pltpu.SemaphoreType.DMA,
        ],
    )
    def body(f1_h, nl_h, h1_h, h2_h, h3_h, idx_v,
             ra1, ra2, ra3, rb1, rb2, rb3, sema, semb):
        wid = lax.axis_index("s") * 2 + lax.axis_index("c")
        base_w = pl.multiple_of(wid * CPW, CPW)
        pltpu.sync_copy(nl_h.at[wid], idx_v)
        bufs_a, bufs_b = (ra1, ra2, ra3), (rb1, rb2, rb3)
        outs = (h1_h, h2_h, h3_h)

        def issue(s, bufs, sem):
            for k in range(3):
                pltpu.async_copy(f1_h.at[idx_v.at[k, s]], bufs[k], sem)

        def drain(s, bufs, sem):
            off = pl.multiple_of(base_w + s * SUB, SUB)
            for k in range(3):
                pltpu.make_async_copy(
                    f1_h.at[idx_v.at[k, s]], bufs[k], sem).wait()
                pltpu.sync_copy(bufs[k], outs[k].at[pl.ds(off, SUB)])

        issue(0, bufs_a, sema)

        def step(k, c):
            s = 2 * k
            issue(s + 1, bufs_b, semb)
            drain(s, bufs_a, sema)

            @pl.when(s + 2 < NSUB)
            def _():
                issue(s + 2, bufs_a, sema)

            drain(s + 1, bufs_b, semb)
            return c

        lax.fori_loop(0, NSUB // 2, step, 0)

    return body(f1, nl3)


def kernel(z_old, W, b, neighbour_list):
    w0, w1, w2, w3 = W[0::4], W[1::4], W[2::4], W[3::4]
    wstack = jnp.concatenate([w0, w1, w2, w3], axis=1)
    wblk = jnp.zeros((2 * D, D), jnp.float32)
    wblk = wblk.at[:D, :64].set(wstack).at[D:, 64:].set(wstack)
    b128 = jnp.tile(b, 8).reshape(1, D)
    eye8 = jnp.eye(8, dtype=jnp.float32)
    kmats = [jnp.kron(eye8, wk[:DD]) for wk in (w0, w1, w2, w3)]
    nlT = jnp.zeros((3, NPAD), jnp.int32).at[:, :N].set(neighbour_list.T)
    q = nlT % PBLK
    nlv = 8 * (BLKR * (nlT // PBLK) + q % BLKR) + q // BLKR
    nl3 = jnp.transpose(nlv.reshape(3, NW, 8, BLKR),
                        (1, 0, 3, 2)).reshape(NW, 3, NSUB, SUB)

    flat = lambda a: jnp.reshape(a, (NPAD, DD))
    wide = lambda a: jnp.reshape(a, (NROW, D))
    p0, p1, p2, p3 = _project(z_old, wblk, b128)
    f1 = _gather_sum(flat(p0), flat(p1), flat(p2), flat(p3), nl3)
    h1, h2, h3 = _gather3(f1, nl3)
    return _final(z_old, wide(f1), wide(h1), wide(h2), wide(h3), kmats)
